# baseline (speedup 1.0000x reference)
import functools

import jax
import jax.numpy as jnp
from jax import lax
from jax.experimental import pallas as pl
from jax.experimental.pallas import tpu as pltpu
from jax.experimental.pallas import tpu_sc as plsc

B = 16384
D = 32
L = 16
NC = 2
NS = 16
NW = NC * NS
B_PER_W = B // NW
G = 4
NGRP = B_PER_W // G
NCG = D // 8


def _make_kernel():
    mesh = plsc.VectorSubcoreMesh(core_axis_name="c", subcore_axis_name="s")

    @functools.partial(
        pl.kernel,
        mesh=mesh,
        out_type=jax.ShapeDtypeStruct((B,), jnp.float32),
        compiler_params=pltpu.CompilerParams(needs_layout_passes=False),
        scratch_types=[
            pltpu.VMEM((B_PER_W,), jnp.int32),
            pltpu.VMEM((B_PER_W,), jnp.int32),
            pltpu.VMEM((2, G * D, 128), jnp.float32),
            pltpu.VMEM((2, G * D, 128), jnp.float32),
            pltpu.VMEM((B_PER_W * D,), jnp.float32),
            pltpu.VMEM((B_PER_W * D,), jnp.float32),
            pltpu.VMEM((B_PER_W,), jnp.float32),
            pltpu.SemaphoreType.DMA,
            pltpu.SemaphoreType.DMA,
            pltpu.SemaphoreType.DMA,
            pltpu.SemaphoreType.DMA,
        ],
    )
    def k(uidx_hbm, iidx_hbm, utab_hbm, itab_hbm, out_hbm,
          uidx_v, iidx_v, utiles_v, itiles_v,
          urows_v, irows_v, out_v,
          usem0, usem1, isem0, isem1):
        wid = lax.axis_index("s") * NC + lax.axis_index("c")
        base = wid * B_PER_W
        pltpu.sync_copy(uidx_hbm.at[pl.ds(base, B_PER_W)], uidx_v)
        pltpu.sync_copy(iidx_hbm.at[pl.ds(base, B_PER_W)], iidx_v)
        usems = [usem0, usem1]
        isems = [isem0, isem1]
        lanes = lax.iota(jnp.int32, L)

        def sread(vec_ref, j):
            chunk = vec_ref[pl.ds((j // L) * L, L)]
            return jnp.sum(jnp.where(lanes == j % L, chunk, 0))

        def fire(grp, buf):
            for g in range(G):
                b = grp * G + g
                ur = sread(uidx_v, b)
                ir = sread(iidx_v, b)
                uoff = pl.multiple_of((ur >> 7) * 128, 128)
                ioff = pl.multiple_of((ir >> 7) * 128, 128)
                pltpu.async_copy(
                    utab_hbm.at[:, pl.ds(uoff, 128)],
                    utiles_v.at[buf, pl.ds(g * D, D), :],
                    usems[buf],
                )
                pltpu.async_copy(
                    itab_hbm.at[:, pl.ds(ioff, 128)],
                    itiles_v.at[buf, pl.ds(g * D, D), :],
                    isems[buf],
                )

        def drain(buf):
            for _ in range(G):
                pltpu.make_async_copy(
                    utab_hbm.at[:, pl.ds(0, 128)],
                    utiles_v.at[buf, pl.ds(0, D), :],
                    usems[buf],
                ).wait()
                pltpu.make_async_copy(
                    itab_hbm.at[:, pl.ds(0, 128)],
                    itiles_v.at[buf, pl.ds(0, D), :],
                    isems[buf],
                ).wait()

        def extract(grp, buf):
            bufv = jnp.full((L,), buf, jnp.int32)
            for g in range(G):
                b = grp * G + g
                ul = jnp.full((L,), sread(uidx_v, b) & 127, jnp.int32)
                il = jnp.full((L,), sread(iidx_v, b) & 127, jnp.int32)
                for h in range(2):
                    rows = g * D + h * L + lanes
                    u = plsc.load_gather(utiles_v, [bufv, rows, ul])
                    iv = plsc.load_gather(itiles_v, [bufv, rows, il])
                    urows_v[pl.ds(b * D + h * L, L)] = u
                    irows_v[pl.ds(b * D + h * L, L)] = iv

        fire(0, 0)

        def pipe(kk, carry):
            fire(2 * kk + 1, 1)
            drain(0)
            extract(2 * kk, 0)

            @pl.when(kk < NGRP // 2 - 1)
            def _():
                fire(2 * kk + 2, 0)

            drain(1)
            extract(2 * kk + 1, 1)
            return carry

        lax.fori_loop(0, NGRP // 2, pipe, 0)

        def body(g, carry):
            flat0 = (g * L + lanes) * D
            acc = jnp.zeros((L,), jnp.float32)
            for d in range(D):
                u = plsc.load_gather(urows_v, [flat0 + d])
                iv = plsc.load_gather(irows_v, [flat0 + d])
                acc = acc + u * iv
            out_v[pl.ds(g * L, L)] = acc
            return carry

        lax.fori_loop(0, B_PER_W // L, body, 0)
        pltpu.sync_copy(out_v, out_hbm.at[pl.ds(base, B_PER_W)])

    return k


_pmf_kernel = _make_kernel()


def kernel(users_index, items_index, user_table, item_table):
    return _pmf_kernel(
        users_index.astype(jnp.int32),
        items_index.astype(jnp.int32),
        user_table.T,
        item_table.T,
    )


# speedup vs baseline: 1.0045x; 1.0045x over previous
import functools

import jax
import jax.numpy as jnp
from jax import lax
from jax.experimental import pallas as pl
from jax.experimental.pallas import tpu as pltpu
from jax.experimental.pallas import tpu_sc as plsc

B = 16384
D = 32
L = 16
NC = 2
NS = 16
NW = NC * NS
B_PER_W = B // NW
G = 4
NGRP = B_PER_W // G
NCG = D // 8


def _make_kernel():
    mesh = plsc.VectorSubcoreMesh(core_axis_name="c", subcore_axis_name="s")

    @functools.partial(
        pl.kernel,
        mesh=mesh,
        out_type=jax.ShapeDtypeStruct((B,), jnp.float32),
        compiler_params=pltpu.CompilerParams(
            needs_layout_passes=False, skip_device_barrier=True
        ),
        scratch_types=[
            pltpu.VMEM((B_PER_W,), jnp.int32),
            pltpu.VMEM((B_PER_W,), jnp.int32),
            pltpu.VMEM((2, G * D, 128), jnp.float32),
            pltpu.VMEM((2, G * D, 128), jnp.float32),
            pltpu.VMEM((B_PER_W * D,), jnp.float32),
            pltpu.VMEM((B_PER_W * D,), jnp.float32),
            pltpu.VMEM((B_PER_W,), jnp.float32),
            pltpu.SemaphoreType.DMA,
            pltpu.SemaphoreType.DMA,
            pltpu.SemaphoreType.DMA,
            pltpu.SemaphoreType.DMA,
        ],
    )
    def k(uidx_hbm, iidx_hbm, utab_hbm, itab_hbm, out_hbm,
          uidx_v, iidx_v, utiles_v, itiles_v,
          urows_v, irows_v, out_v,
          usem0, usem1, isem0, isem1):
        wid = lax.axis_index("s") * NC + lax.axis_index("c")
        base = wid * B_PER_W
        pltpu.sync_copy(uidx_hbm.at[pl.ds(base, B_PER_W)], uidx_v)
        pltpu.sync_copy(iidx_hbm.at[pl.ds(base, B_PER_W)], iidx_v)
        usems = [usem0, usem1]
        isems = [isem0, isem1]
        lanes = lax.iota(jnp.int32, L)

        def sread(vec_ref, j):
            chunk = vec_ref[pl.ds((j // L) * L, L)]
            return jnp.sum(jnp.where(lanes == j % L, chunk, 0))

        def fire(grp, buf):
            for g in range(G):
                b = grp * G + g
                ur = sread(uidx_v, b)
                ir = sread(iidx_v, b)
                uoff = pl.multiple_of((ur >> 7) * 128, 128)
                ioff = pl.multiple_of((ir >> 7) * 128, 128)
                pltpu.async_copy(
                    utab_hbm.at[:, pl.ds(uoff, 128)],
                    utiles_v.at[buf, pl.ds(g * D, D), :],
                    usems[buf],
                )
                pltpu.async_copy(
                    itab_hbm.at[:, pl.ds(ioff, 128)],
                    itiles_v.at[buf, pl.ds(g * D, D), :],
                    isems[buf],
                )

        def drain(buf):
            for _ in range(G):
                pltpu.make_async_copy(
                    utab_hbm.at[:, pl.ds(0, 128)],
                    utiles_v.at[buf, pl.ds(0, D), :],
                    usems[buf],
                ).wait()
                pltpu.make_async_copy(
                    itab_hbm.at[:, pl.ds(0, 128)],
                    itiles_v.at[buf, pl.ds(0, D), :],
                    isems[buf],
                ).wait()

        def extract(grp, buf):
            bufv = jnp.full((L,), buf, jnp.int32)
            for g in range(G):
                b = grp * G + g
                ul = jnp.full((L,), sread(uidx_v, b) & 127, jnp.int32)
                il = jnp.full((L,), sread(iidx_v, b) & 127, jnp.int32)
                for h in range(2):
                    rows = g * D + h * L + lanes
                    u = plsc.load_gather(utiles_v, [bufv, rows, ul])
                    iv = plsc.load_gather(itiles_v, [bufv, rows, il])
                    urows_v[pl.ds(b * D + h * L, L)] = u
                    irows_v[pl.ds(b * D + h * L, L)] = iv

        fire(0, 0)

        def pipe(kk, carry):
            fire(2 * kk + 1, 1)
            drain(0)
            extract(2 * kk, 0)

            @pl.when(kk < NGRP // 2 - 1)
            def _():
                fire(2 * kk + 2, 0)

            drain(1)
            extract(2 * kk + 1, 1)
            return carry

        lax.fori_loop(0, NGRP // 2, pipe, 0)

        def body(g, carry):
            flat0 = (g * L + lanes) * D
            acc = jnp.zeros((L,), jnp.float32)
            for d in range(D):
                u = plsc.load_gather(urows_v, [flat0 + d])
                iv = plsc.load_gather(irows_v, [flat0 + d])
                acc = acc + u * iv
            out_v[pl.ds(g * L, L)] = acc
            return carry

        lax.fori_loop(0, B_PER_W // L, body, 0)
        pltpu.sync_copy(out_v, out_hbm.at[pl.ds(base, B_PER_W)])

    return k


_pmf_kernel = _make_kernel()


def kernel(users_index, items_index, user_table, item_table):
    return _pmf_kernel(
        users_index.astype(jnp.int32),
        items_index.astype(jnp.int32),
        user_table.T,
        item_table.T,
    )
